# Initial kernel scaffold; baseline (speedup 1.0000x reference)
import jax
import jax.numpy as jnp
from jax import lax
from jax.experimental import pallas as pl
from jax.experimental.pallas import tpu as pltpu
from jax.experimental.pallas import tpu_sc as plsc

N_NODES = 10000
N_EDGES = 160000
D = 256

L = 16
NC = 2
NS = 16
NW = NC * NS
RPW = 313
CE = 3200
G = 32
RC = 32
NEG_HUGE = -3e38


def _mm_body(x_ref, wt_ref, wp_ref, y_ref, z_ref):
    xb = x_ref[...]
    dn = (((1,), (1,)), ((), ()))
    y_ref[...] = lax.dot_general(xb, wt_ref[...], dn,
                                 preferred_element_type=jnp.float32)
    z_ref[...] = lax.dot_general(xb, wp_ref[...], dn,
                                 preferred_element_type=jnp.float32)


def _matmuls(x, W_theta, W_phi):
    R = 2000
    grid = (N_NODES // R,)
    return pl.pallas_call(
        _mm_body,
        grid=grid,
        in_specs=[
            pl.BlockSpec((R, D), lambda i: (i, 0)),
            pl.BlockSpec((D, D), lambda i: (0, 0)),
            pl.BlockSpec((D, D), lambda i: (0, 0)),
        ],
        out_specs=[
            pl.BlockSpec((R, D), lambda i: (i, 0)),
            pl.BlockSpec((R, D), lambda i: (i, 0)),
        ],
        out_shape=[
            jax.ShapeDtypeStruct((N_NODES, D), jnp.float32),
            jax.ShapeDtypeStruct((N_NODES, D), jnp.float32),
        ],
    )(x, W_theta, W_phi)


def _sc_body(y_hbm, z_hbm, row_hbm, col_hbm, out_hbm,
             acc, colbuf, rowbuf, dlist, ilist, staged, ybuf, zbuf, obuf,
             sem):
    c = lax.axis_index("c")
    s = lax.axis_index("s")
    wid = s * NC + c
    lo = wid * RPW

    iota16 = lax.iota(jnp.int32, 16)
    neg_inf = jnp.full((L,), -jnp.inf, jnp.float32)

    def init_body(i, _):
        acc[pl.ds(i * L, L)] = neg_inf
        return 0
    lax.fori_loop(0, (RPW + 1) * D // L, init_body, 0)

    def chunk_body(ci, _):
        e0 = ci * CE
        pltpu.sync_copy(col_hbm.at[pl.ds(e0, CE)], colbuf)
        pltpu.sync_copy(row_hbm.at[pl.ds(e0, CE)], rowbuf)

        def scan_body(g, off):
            cv = colbuf[pl.ds(g * L, L)]
            rv = rowbuf[pl.ds(g * L, L)]
            m = (cv >= lo) & (cv < lo + RPW)
            plsc.store_compressed(dlist.at[pl.ds(off, L)], cv - lo, mask=m)
            plsc.store_compressed(ilist.at[pl.ds(off, L)], rv, mask=m)
            return off + jnp.sum(m.astype(jnp.int32))
        off = lax.fori_loop(0, CE // L, scan_body, 0)

        trash = jnp.full((L,), RPW, jnp.int32)
        zero = jnp.zeros((L,), jnp.int32)
        dlist[pl.ds(off, L)] = trash
        dlist[pl.ds(off + L, L)] = trash
        ilist[pl.ds(off, L)] = zero
        ilist[pl.ds(off + L, L)] = zero

        nb = (off + G - 1) // G

        def batch_body(b, _):
            idx_sl = ilist.at[pl.ds(b * G, G)]
            pltpu.async_copy(y_hbm.at[idx_sl], staged, sem).wait()

            def grp_body(g2, _):
                r0 = g2 * L
                dv = dlist[pl.ds(b * G + r0, L)]
                for j in range(L):
                    dj = jnp.broadcast_to(dv[j], (L,))
                    base = dj * D
                    rj = jnp.broadcast_to(r0 + j, (L,))
                    for v in range(D // L):
                        cidx = v * L + iota16
                        a = plsc.load_gather(acc, [base + cidx])
                        sv = plsc.load_gather(staged, [rj, cidx])
                        plsc.store_scatter(acc, [base + cidx],
                                           jnp.maximum(a, sv))
                return 0
            lax.fori_loop(0, G // L, grp_body, 0)
            return 0
        lax.fori_loop(0, nb, batch_body, 0)
        return 0
    lax.fori_loop(0, N_EDGES // CE, chunk_body, 0)

    cap = jnp.minimum(lo + (RPW - RC), N_NODES - RC)
    nrc = (RPW + RC - 1) // RC

    def comb_body(rb, _):
        start = jnp.minimum(lo + rb * RC, cap)
        local = start - lo
        pltpu.sync_copy(y_hbm.at[pl.ds(start, RC)], ybuf)
        pltpu.sync_copy(z_hbm.at[pl.ds(start, RC)], zbuf)

        def row_body(r, _):
            rvec = jnp.broadcast_to(r, (L,))
            for v in range(D // L):
                cidx = v * L + iota16
                a = acc[pl.ds((local + r) * D + v * L, L)]
                yv = plsc.load_gather(ybuf, [rvec, cidx])
                zv = plsc.load_gather(zbuf, [rvec, cidx])
                ov = zv + jnp.where(a > NEG_HUGE, a - yv, 0.0)
                plsc.store_scatter(obuf, [rvec, cidx], ov)
            return 0
        lax.fori_loop(0, RC, row_body, 0)
        pltpu.sync_copy(obuf, out_hbm.at[pl.ds(start, RC)])
        return 0
    lax.fori_loop(0, nrc, comb_body, 0)


def _segmax_combine(y, z, row, col):
    mesh = plsc.VectorSubcoreMesh(core_axis_name="c", subcore_axis_name="s")
    f = pl.kernel(
        _sc_body,
        out_type=jax.ShapeDtypeStruct((N_NODES, D), jnp.float32),
        mesh=mesh,
        scratch_types=[
            pltpu.VMEM(((RPW + 1) * D,), jnp.float32),
            pltpu.VMEM((CE,), jnp.int32),
            pltpu.VMEM((CE,), jnp.int32),
            pltpu.VMEM((CE + 2 * G,), jnp.int32),
            pltpu.VMEM((CE + 2 * G,), jnp.int32),
            pltpu.VMEM((G, D), jnp.float32),
            pltpu.VMEM((RC, D), jnp.float32),
            pltpu.VMEM((RC, D), jnp.float32),
            pltpu.VMEM((RC, D), jnp.float32),
            pltpu.SemaphoreType.DMA,
        ],
    )
    return f(y, z, row, col)


def kernel(x, edge_index, W_theta, W_phi):
    row = edge_index[0]
    col = edge_index[1]
    y, z = _matmuls(x, W_theta, W_phi)
    return _segmax_combine(y, z, row, col)


# speedup vs baseline: 1.0865x; 1.0865x over previous
import jax
import jax.numpy as jnp
from jax import lax
from jax.experimental import pallas as pl
from jax.experimental.pallas import tpu as pltpu
from jax.experimental.pallas import tpu_sc as plsc

N_NODES = 10000
N_EDGES = 160000
D = 256

L = 16
NC = 2
NS = 16
NW = NC * NS
RPW = 320
CE = 3200
G = 32
RC = 32
NEG_HUGE = -3e38


def _mm_body(x_ref, wt_ref, wp_ref, y_ref, z_ref):
    xb = x_ref[...]
    dn = (((1,), (1,)), ((), ()))
    y_ref[...] = lax.dot_general(xb, wt_ref[...], dn,
                                 preferred_element_type=jnp.float32)
    z_ref[...] = lax.dot_general(xb, wp_ref[...], dn,
                                 preferred_element_type=jnp.float32)


def _matmuls(x, W_theta, W_phi):
    R = 2000
    grid = (N_NODES // R,)
    return pl.pallas_call(
        _mm_body,
        grid=grid,
        in_specs=[
            pl.BlockSpec((R, D), lambda i: (i, 0)),
            pl.BlockSpec((D, D), lambda i: (0, 0)),
            pl.BlockSpec((D, D), lambda i: (0, 0)),
        ],
        out_specs=[
            pl.BlockSpec((R, D), lambda i: (i, 0)),
            pl.BlockSpec((R, D), lambda i: (i, 0)),
        ],
        out_shape=[
            jax.ShapeDtypeStruct((N_NODES, D), jnp.float32),
            jax.ShapeDtypeStruct((N_NODES, D), jnp.float32),
        ],
    )(x, W_theta, W_phi)


def _sc_body(y_hbm, z_hbm, row_hbm, col_hbm, out_hbm,
             acc, colbuf, rowbuf, dlist, ilist, staged, ybuf, zbuf, obuf,
             sem):
    c = lax.axis_index("c")
    s = lax.axis_index("s")
    wid = s * NC + c
    lo = wid * RPW

    iota16 = lax.iota(jnp.int32, 16)
    neg_inf = jnp.full((L,), -jnp.inf, jnp.float32)

    def init_body(i, _):
        acc[pl.ds(i * L, L)] = neg_inf
        return 0
    lax.fori_loop(0, (RPW + 1) * D // L, init_body, 0)

    def chunk_body(ci, _):
        e0 = pl.multiple_of(ci * CE, CE)
        pltpu.sync_copy(col_hbm.at[pl.ds(e0, CE)], colbuf)
        pltpu.sync_copy(row_hbm.at[pl.ds(e0, CE)], rowbuf)

        def scan_body(g, off):
            cv = colbuf[pl.ds(g * L, L)]
            rv = rowbuf[pl.ds(g * L, L)]
            m = (cv >= lo) & (cv < lo + RPW)
            cs = plsc.cumsum(m.astype(jnp.int32))
            pos = off + cs - 1
            plsc.store_scatter(dlist, [pos], cv - lo, mask=m)
            plsc.store_scatter(ilist, [pos], rv, mask=m)
            return off + cs[L - 1]
        off = lax.fori_loop(0, CE // L, scan_body, 0)

        trash = jnp.full((L,), RPW, jnp.int32)
        zero = jnp.zeros((L,), jnp.int32)
        dlist[pl.ds(off, L)] = trash
        dlist[pl.ds(off + L, L)] = trash
        ilist[pl.ds(off, L)] = zero
        ilist[pl.ds(off + L, L)] = zero

        nb = (off + G - 1) // G

        def batch_body(b, _):
            idx_sl = ilist.at[pl.ds(pl.multiple_of(b * G, G), G)]
            pltpu.async_copy(y_hbm.at[idx_sl], staged, sem).wait()

            def grp_body(g2, _):
                r0 = g2 * L
                dv = dlist[pl.ds(b * G + r0, L)]
                for j in range(L):
                    dj = jnp.broadcast_to(dv[j], (L,))
                    base = dj * D
                    rj = jnp.broadcast_to(r0 + j, (L,))
                    for v in range(D // L):
                        cidx = v * L + iota16
                        a = plsc.load_gather(acc, [base + cidx])
                        sv = plsc.load_gather(staged, [rj, cidx])
                        plsc.store_scatter(acc, [base + cidx],
                                           jnp.maximum(a, sv))
                return 0
            lax.fori_loop(0, G // L, grp_body, 0)
            return 0
        lax.fori_loop(0, nb, batch_body, 0)
        return 0
    lax.fori_loop(0, N_EDGES // CE, chunk_body, 0)

    cap = jnp.minimum(lo + (RPW - RC), N_NODES - RC)
    nrc = (RPW + RC - 1) // RC

    def comb_body(rb, _):
        start = pl.multiple_of(jnp.minimum(lo + rb * RC, cap), 8)
        local = start - lo
        pltpu.sync_copy(y_hbm.at[pl.ds(start, RC)], ybuf)
        pltpu.sync_copy(z_hbm.at[pl.ds(start, RC)], zbuf)

        def row_body(r, _):
            rvec = jnp.broadcast_to(r, (L,))
            for v in range(D // L):
                cidx = v * L + iota16
                a = acc[pl.ds((local + r) * D + v * L, L)]
                yv = plsc.load_gather(ybuf, [rvec, cidx])
                zv = plsc.load_gather(zbuf, [rvec, cidx])
                ov = zv + jnp.where(a > NEG_HUGE, a - yv, 0.0)
                plsc.store_scatter(obuf, [rvec, cidx], ov)
            return 0
        lax.fori_loop(0, RC, row_body, 0)
        pltpu.sync_copy(obuf, out_hbm.at[pl.ds(start, RC)])
        return 0
    lax.fori_loop(0, nrc, comb_body, 0)


def _segmax_combine(y, z, row, col):
    mesh = plsc.VectorSubcoreMesh(core_axis_name="c", subcore_axis_name="s",
                                  num_cores=NC, num_subcores=NS)
    f = pl.kernel(
        _sc_body,
        out_type=jax.ShapeDtypeStruct((N_NODES, D), jnp.float32),
        mesh=mesh,
        compiler_params=pltpu.CompilerParams(needs_layout_passes=False),
        scratch_types=[
            pltpu.VMEM(((RPW + 1) * D,), jnp.float32),
            pltpu.VMEM((CE,), jnp.int32),
            pltpu.VMEM((CE,), jnp.int32),
            pltpu.VMEM((CE + 2 * G,), jnp.int32),
            pltpu.VMEM((CE + 2 * G,), jnp.int32),
            pltpu.VMEM((G, D), jnp.float32),
            pltpu.VMEM((RC, D), jnp.float32),
            pltpu.VMEM((RC, D), jnp.float32),
            pltpu.VMEM((RC, D), jnp.float32),
            pltpu.SemaphoreType.DMA,
        ],
    )
    return f(y, z, row, col)


def kernel(x, edge_index, W_theta, W_phi):
    row = edge_index[0]
    col = edge_index[1]
    y, z = _matmuls(x, W_theta, W_phi)
    return _segmax_combine(y, z, row, col)
